# baseline (speedup 1.0000x reference)
import jax
import jax.numpy as jnp
from jax import lax
from jax.experimental import pallas as pl
from jax.experimental.pallas import tpu as pltpu

H, L, D = 12, 2048, 64
BLK = 64
NBLK = L // BLK
TOPK = 4
SCALE = 1.0 / 8.0
DE = 2 * D
KSEL = TOPK * BLK


NCH = 2
CBLK = NBLK // NCH


def _fused_kernel(idx_ref, q_ref, k_ref, v_ref, w_ref, b_ref, o_ref,
                  ck_ref, vext_ref, sext_ref, st_ref, p_ref, oext_ref,
                  ne_ref):
    h = pl.program_id(0)
    ii = pl.program_id(1)

    @pl.when(ii == 0)
    def _stats():
        kx = k_ref[0]
        ek = jnp.exp(kx)
        ck_ref[...] = ek / jnp.sum(ek, axis=-1, keepdims=True)
        ones_blk = jnp.ones((BLK, D), jnp.float32)
        for j in range(NBLK):
            vext_ref[j, :, :D] = v_ref[0, j * BLK:(j + 1) * BLK, :]
            vext_ref[j, :, D:] = ones_blk
        parts = [jnp.zeros((D, DE), jnp.float32) for _ in range(4)]
        for j in range(NBLK):
            ckb = ck_ref[j * BLK:(j + 1) * BLK, :]
            sext = lax.dot_general(ckb, vext_ref[j], (((0,), (0,)), ((), ())),
                                   preferred_element_type=jnp.float32)
            sext_ref[j] = sext
            parts[j % 4] = parts[j % 4] + sext
        st_ref[...] = (parts[0] + parts[1]) + (parts[2] + parts[3])

    st = st_ref[...]
    sel = [[idx_ref[h, ii * CBLK + i, t] for t in range(TOPK)]
           for i in range(CBLK)]

    for i in range(CBLK):
        qb = q_ref[0, i * BLK:(i + 1) * BLK, :]
        for t, bj in enumerate(sel[i]):
            kj = k_ref[0, pl.ds(bj * BLK, BLK), :]
            lg = lax.dot_general(qb, kj, (((1,), (1,)), ((), ())),
                                 preferred_element_type=jnp.float32) * SCALE
            p_ref[i, :, t * BLK:(t + 1) * BLK] = jnp.exp(lg)

    eq = jnp.exp(q_ref[0])
    for i in range(CBLK):
        bs = sel[i]
        vs = [vext_ref[pl.ds(bj, 1)][0] for bj in bs]
        ss = [sext_ref[pl.ds(bj, 1)][0] for bj in bs]
        o01 = jnp.dot(p_ref[i, :, 0:BLK], vs[0],
                      preferred_element_type=jnp.float32) + \
              jnp.dot(p_ref[i, :, BLK:2 * BLK], vs[1],
                      preferred_element_type=jnp.float32)
        o23 = jnp.dot(p_ref[i, :, 2 * BLK:3 * BLK], vs[2],
                      preferred_element_type=jnp.float32) + \
              jnp.dot(p_ref[i, :, 3 * BLK:4 * BLK], vs[3],
                      preferred_element_type=jnp.float32)
        oext_ref[i * BLK:(i + 1) * BLK, :] = o01 + o23
        sc = st - ((ss[0] + ss[1]) + (ss[2] + ss[3]))
        ne_ref[i * BLK:(i + 1) * BLK, :] = jnp.dot(
            eq[i * BLK:(i + 1) * BLK, :], sc,
            preferred_element_type=jnp.float32)

    oe = oext_ref[...]
    ne = ne_ref[...]
    rsq = jnp.sum(eq, axis=-1, keepdims=True)
    o_s = oe[:, :D] / oe[:, D:D + 1]
    o_l = ne[:, :D] / (ne[:, D:D + 1] + 1e-6 * rsq)
    o_ref[0] = o_s + lax.dot_general(
        o_l, w_ref[...], (((1,), (1,)), ((), ())),
        preferred_element_type=jnp.float32) + b_ref[...]


@jax.jit
def kernel(q, k, v, W_l, b_l):
    qh = q.reshape(H, L, D)
    kh = k.reshape(H, L, D)
    vh = v.reshape(H, L, D)

    qp = q.reshape(1, H, NBLK, BLK, D).mean(axis=3)
    kp = k.reshape(1, H, NBLK, BLK, D).mean(axis=3)
    scores = jnp.einsum('bhqd,bhkd->bhqk', qp, kp)
    s_work = scores.reshape(H * NBLK, NBLK)
    lanes = jnp.arange(NBLK, dtype=jnp.int32)[None, :]
    cols = []
    for _ in range(TOPK):
        a = jnp.argmax(s_work, axis=-1).astype(jnp.int32)
        cols.append(a)
        s_work = jnp.where(lanes == a[:, None], -jnp.inf, s_work)
    idx = jnp.stack(cols, axis=-1).reshape(H, NBLK, TOPK)

    out = _run(idx, qh, kh, vh, W_l, b_l)
    return out.reshape(1, H, L, D)


def _run(idx, qh, kh, vh, W_l, b_l):
    LC = L // NCH
    return pl.pallas_call(
        _fused_kernel,
        grid=(H, NCH),
        in_specs=[
            pl.BlockSpec(memory_space=pltpu.SMEM),
            pl.BlockSpec((1, LC, D), lambda h, i: (h, i, 0)),
            pl.BlockSpec((1, L, D), lambda h, i: (h, 0, 0)),
            pl.BlockSpec((1, L, D), lambda h, i: (h, 0, 0)),
            pl.BlockSpec((D, D), lambda h, i: (0, 0)),
            pl.BlockSpec((1, D), lambda h, i: (0, 0)),
        ],
        out_specs=pl.BlockSpec((1, LC, D), lambda h, i: (h, i, 0)),
        out_shape=jax.ShapeDtypeStruct((H, L, D), jnp.float32),
        scratch_shapes=[
            pltpu.VMEM((L, D), jnp.float32),
            pltpu.VMEM((NBLK, BLK, DE), jnp.float32),
            pltpu.VMEM((NBLK, D, DE), jnp.float32),
            pltpu.VMEM((D, DE), jnp.float32),
            pltpu.VMEM((CBLK, BLK, KSEL), jnp.float32),
            pltpu.VMEM((L // NCH, DE), jnp.float32),
            pltpu.VMEM((L // NCH, DE), jnp.float32),
        ],
        compiler_params=pltpu.CompilerParams(
            dimension_semantics=("arbitrary", "arbitrary")),
    )(idx, qh, kh, vh, W_l, b_l.reshape(1, D))


# speedup vs baseline: 1.0795x; 1.0795x over previous
import jax
import jax.numpy as jnp
from jax import lax
from jax.experimental import pallas as pl
from jax.experimental.pallas import tpu as pltpu

H, L, D = 12, 2048, 64
BLK = 64
NBLK = L // BLK
TOPK = 4
SCALE = 1.0 / 8.0
DE = 2 * D
KSEL = TOPK * BLK


def _fused_kernel(idx_ref, q_ref, k_ref, v_ref, w_ref, b_ref, o_ref,
                  ck_ref, eq_ref, vext_ref, sext_ref, p_ref, oext_ref,
                  ne_ref):
    h = pl.program_id(0)

    kx = k_ref[0]
    ek = jnp.exp(kx)
    ck_ref[...] = ek / jnp.sum(ek, axis=-1, keepdims=True)
    eq_ref[...] = jnp.exp(q_ref[0])
    ones_blk = jnp.ones((BLK, D), jnp.float32)
    for j in range(NBLK):
        vext_ref[j, :, :D] = v_ref[0, j * BLK:(j + 1) * BLK, :]
        vext_ref[j, :, D:] = ones_blk

    parts = [jnp.zeros((D, DE), jnp.float32) for _ in range(4)]
    for j in range(NBLK):
        ckb = ck_ref[j * BLK:(j + 1) * BLK, :]
        sext = lax.dot_general(ckb, vext_ref[j], (((0,), (0,)), ((), ())),
                               preferred_element_type=jnp.float32)
        sext_ref[j] = sext
        parts[j % 4] = parts[j % 4] + sext
    st = (parts[0] + parts[1]) + (parts[2] + parts[3])

    sel = [[idx_ref[h, i, t] for t in range(TOPK)] for i in range(NBLK)]

    for i in range(NBLK):
        qb = q_ref[0, i * BLK:(i + 1) * BLK, :]
        for t, bj in enumerate(sel[i]):
            kj = k_ref[0, pl.ds(bj * BLK, BLK), :]
            lg = lax.dot_general(qb, kj, (((1,), (1,)), ((), ())),
                                 preferred_element_type=jnp.float32) * SCALE
            p_ref[i, :, t * BLK:(t + 1) * BLK] = jnp.exp(lg)

    for i in range(NBLK):
        bs = sel[i]
        vs = [vext_ref[pl.ds(bj, 1)][0] for bj in bs]
        ss = [sext_ref[pl.ds(bj, 1)][0] for bj in bs]
        o01 = jnp.dot(p_ref[i, :, 0:BLK], vs[0],
                      preferred_element_type=jnp.float32) + \
              jnp.dot(p_ref[i, :, BLK:2 * BLK], vs[1],
                      preferred_element_type=jnp.float32)
        o23 = jnp.dot(p_ref[i, :, 2 * BLK:3 * BLK], vs[2],
                      preferred_element_type=jnp.float32) + \
              jnp.dot(p_ref[i, :, 3 * BLK:4 * BLK], vs[3],
                      preferred_element_type=jnp.float32)
        oext_ref[i * BLK:(i + 1) * BLK, :] = o01 + o23
        sc = st - ((ss[0] + ss[1]) + (ss[2] + ss[3]))
        ne_ref[i * BLK:(i + 1) * BLK, :] = jnp.dot(
            eq_ref[i * BLK:(i + 1) * BLK, :], sc,
            preferred_element_type=jnp.float32)

    oe = oext_ref[...]
    ne = ne_ref[...]
    eq = eq_ref[...]
    rsq = jnp.sum(eq, axis=-1, keepdims=True)
    o_s = oe[:, :D] / oe[:, D:D + 1]
    o_l = ne[:, :D] / (ne[:, D:D + 1] + 1e-6 * rsq)
    o_ref[0] = o_s + lax.dot_general(
        o_l, w_ref[...], (((1,), (1,)), ((), ())),
        preferred_element_type=jnp.float32) + b_ref[...]


@jax.jit
def kernel(q, k, v, W_l, b_l):
    qh = q.reshape(H, L, D)
    kh = k.reshape(H, L, D)
    vh = v.reshape(H, L, D)

    qp = q.reshape(1, H, NBLK, BLK, D).mean(axis=3)
    kp = k.reshape(1, H, NBLK, BLK, D).mean(axis=3)
    scores = jnp.einsum('bhqd,bhkd->bhqk', qp, kp)
    s_work = scores.reshape(H * NBLK, NBLK)
    lanes = jnp.arange(NBLK, dtype=jnp.int32)[None, :]
    cols = []
    for _ in range(TOPK):
        a = jnp.argmax(s_work, axis=-1).astype(jnp.int32)
        cols.append(a)
        s_work = jnp.where(lanes == a[:, None], -jnp.inf, s_work)
    idx = jnp.stack(cols, axis=-1).reshape(H, NBLK, TOPK)

    out = _run(idx, qh, kh, vh, W_l, b_l)
    return out.reshape(1, H, L, D)


def _run(idx, qh, kh, vh, W_l, b_l):
    return pl.pallas_call(
        _fused_kernel,
        grid=(H,),
        in_specs=[
            pl.BlockSpec(memory_space=pltpu.SMEM),
            pl.BlockSpec((1, L, D), lambda h: (h, 0, 0)),
            pl.BlockSpec((1, L, D), lambda h: (h, 0, 0)),
            pl.BlockSpec((1, L, D), lambda h: (h, 0, 0)),
            pl.BlockSpec((D, D), lambda h: (0, 0)),
            pl.BlockSpec((1, D), lambda h: (0, 0)),
        ],
        out_specs=pl.BlockSpec((1, L, D), lambda h: (h, 0, 0)),
        out_shape=jax.ShapeDtypeStruct((H, L, D), jnp.float32),
        scratch_shapes=[
            pltpu.VMEM((L, D), jnp.float32),
            pltpu.VMEM((L, D), jnp.float32),
            pltpu.VMEM((NBLK, BLK, DE), jnp.float32),
            pltpu.VMEM((NBLK, D, DE), jnp.float32),
            pltpu.VMEM((NBLK, BLK, KSEL), jnp.float32),
            pltpu.VMEM((L, DE), jnp.float32),
            pltpu.VMEM((L, DE), jnp.float32),
        ],
        compiler_params=pltpu.CompilerParams(
            dimension_semantics=("arbitrary",)),
    )(idx, qh, kh, vh, W_l, b_l.reshape(1, D))
